# baseline (speedup 1.0000x reference)
import functools

import jax
import jax.numpy as jnp
from jax import lax
from jax.experimental import pallas as pl
from jax.experimental.pallas import tpu as pltpu
from jax.experimental.pallas import tpu_sc as plsc

_R = 4096
_C = 200
_D = 64
_NW = 32
_RPW = _R // _NW
_G = 4
_NGRP = _RPW // _G
_NIT = _NGRP // 2
_SPLITS = ((0, 128), (128, 72))

_mesh = plsc.VectorSubcoreMesh(core_axis_name="c", subcore_axis_name="s")


@functools.partial(
    pl.kernel,
    mesh=_mesh,
    out_type=jax.ShapeDtypeStruct((_R, _C, _D), jnp.float32),
    scratch_types=[
        pltpu.VMEM((2 * _G, _C), jnp.int32),
        pltpu.VMEM((_G, _C, _D), jnp.float32),
        pltpu.VMEM((_G, _C, _D), jnp.float32),
        pltpu.SemaphoreType.DMA,
        pltpu.SemaphoreType.DMA,
        pltpu.SemaphoreType.DMA,
    ],
    compiler_params=pltpu.CompilerParams(use_tc_tiling_on_sc=False),
)
def _gather_kernel(x_hbm, w_hbm, out_hbm, idx_v, rows0_v, rows1_v,
                   gsem, wsem0, wsem1):
    wid = lax.axis_index("s") * 2 + lax.axis_index("c")
    row0 = wid * _RPW

    def fire(iofs, rows_v):
        copies = []
        for g in range(_G):
            for (lo, n) in _SPLITS:
                copies.append(pltpu.async_copy(
                    w_hbm.at[idx_v.at[iofs + g, pl.ds(lo, n)]],
                    rows_v.at[g, pl.ds(lo, n)],
                    gsem,
                ))
        return copies

    def body(i, carry):
        r_a = row0 + 2 * i * _G
        pltpu.sync_copy(x_hbm.at[pl.ds(r_a, 2 * _G)], idx_v)
        @pl.when(i > 0)
        def _():
            pltpu.make_async_copy(
                rows0_v, out_hbm.at[pl.ds(0, _G)], wsem0).wait()
        ca = fire(0, rows0_v)
        @pl.when(i > 0)
        def _():
            pltpu.make_async_copy(
                rows1_v, out_hbm.at[pl.ds(0, _G)], wsem1).wait()
        for c in ca:
            c.wait()
        pltpu.async_copy(rows0_v, out_hbm.at[pl.ds(r_a, _G)], wsem0)
        cb = fire(_G, rows1_v)
        for c in cb:
            c.wait()
        pltpu.async_copy(rows1_v, out_hbm.at[pl.ds(r_a + _G, _G)], wsem1)
        return carry

    lax.fori_loop(0, _NIT, body, 0)
    pltpu.make_async_copy(rows0_v, out_hbm.at[pl.ds(0, _G)], wsem0).wait()
    pltpu.make_async_copy(rows1_v, out_hbm.at[pl.ds(0, _G)], wsem1).wait()


def kernel(x, weight):
    return _gather_kernel(x.astype(jnp.int32), weight)
